# baseline (speedup 1.0000x reference)
import functools

import jax
import jax.numpy as jnp
from jax import lax
from jax.experimental import pallas as pl
from jax.experimental.pallas import tpu as pltpu
from jax.experimental.pallas import tpu_sc as plsc

EMBED = 32
LANES = 16
SUB = 128


@functools.lru_cache(maxsize=None)
def _make_gather(batch: int, hist: int):
    info = plsc.get_sparse_core_info()
    nc, ns = info.num_cores, info.num_subcores
    nw = nc * ns
    bw = batch // nw
    nsub = bw // SUB
    assert batch % nw == 0 and bw % SUB == 0
    mesh = plsc.VectorSubcoreMesh(core_axis_name="c", subcore_axis_name="s")

    @functools.partial(
        pl.kernel,
        mesh=mesh,
        out_type=jax.ShapeDtypeStruct((hist, EMBED, batch), jnp.float32),
        scratch_types=[
            pltpu.VMEM((bw,), jnp.int32),
            pltpu.VMEM((bw, EMBED), jnp.float32),
            pltpu.VMEM((EMBED, bw), jnp.float32),
            pltpu.SemaphoreType.DMA,
        ],
        compiler_params=pltpu.CompilerParams(
            use_tc_tiling_on_sc=False, needs_layout_passes=False
        ),
    )
    def gather_kernel(table_hbm, idst_hbm, out_hbm, idx_v, rows_v, slab_v,
                      sem):
        wid = lax.axis_index("s") * nc + lax.axis_index("c")
        b0 = wid * bw

        def body(h, carry):
            pltpu.sync_copy(idst_hbm.at[h, pl.ds(b0, bw)], idx_v)
            for j in range(nsub):
                pltpu.async_copy(
                    table_hbm.at[idx_v.at[pl.ds(j * SUB, SUB)]],
                    rows_v.at[pl.ds(j * SUB, SUB), :],
                    sem,
                )
            for j in range(nsub):
                pltpu.make_async_copy(
                    table_hbm.at[idx_v.at[pl.ds(j * SUB, SUB)]],
                    rows_v.at[pl.ds(j * SUB, SUB), :],
                    sem,
                ).wait()

            def tr_body(bg, carry2):
                row_ids = bg * LANES + lax.iota(jnp.int32, LANES)
                for e in range(EMBED):
                    col_ids = jnp.full((LANES,), e, jnp.int32)
                    vals = plsc.load_gather(rows_v, [row_ids, col_ids])
                    slab_v[e, pl.ds(bg * LANES, LANES)] = vals
                return carry2

            lax.fori_loop(0, bw // LANES, tr_body, 0)
            pltpu.sync_copy(slab_v, out_hbm.at[h, :, pl.ds(b0, bw)])
            return carry

        lax.fori_loop(0, hist, body, 0)

    return gather_kernel


def kernel(input_ids, weight):
    batch, hist = input_ids.shape
    ids_t = input_ids.T.astype(jnp.int32)
    out_t = _make_gather(batch, hist)(weight, ids_t)
    return jnp.transpose(out_t, (2, 0, 1))


# speedup vs baseline: 1.3685x; 1.3685x over previous
import functools

import jax
import jax.numpy as jnp
from jax import lax
from jax.experimental import pallas as pl
from jax.experimental.pallas import tpu as pltpu
from jax.experimental.pallas import tpu_sc as plsc

EMBED = 32
LANES = 16
SUB = 128


@functools.lru_cache(maxsize=None)
def _make_gather(batch: int, hist: int):
    info = plsc.get_sparse_core_info()
    nc, ns = info.num_cores, info.num_subcores
    nw = nc * ns
    bw = batch // nw
    nsub = bw // SUB
    assert batch % nw == 0 and bw % SUB == 0
    mesh = plsc.VectorSubcoreMesh(core_axis_name="c", subcore_axis_name="s")

    @functools.partial(
        pl.kernel,
        mesh=mesh,
        out_type=jax.ShapeDtypeStruct((hist, EMBED, batch), jnp.float32),
        scratch_types=[
            pltpu.VMEM((bw,), jnp.int32),
            pltpu.VMEM((bw, EMBED), jnp.float32),
            pltpu.VMEM((EMBED, bw), jnp.float32),
            pltpu.SemaphoreType.DMA,
        ],
        compiler_params=pltpu.CompilerParams(
            use_tc_tiling_on_sc=False, needs_layout_passes=False
        ),
    )
    def gather_kernel(table_hbm, idst_hbm, out_hbm, idx_v, rows_v, slab_v,
                      sem):
        wid = lax.axis_index("s") * nc + lax.axis_index("c")
        b0 = wid * bw

        def body(h, carry):
            pltpu.sync_copy(idst_hbm.at[h, pl.ds(b0, bw)], idx_v)
            for j in range(nsub):
                pltpu.async_copy(
                    table_hbm.at[idx_v.at[pl.ds(j * SUB, SUB)]],
                    rows_v.at[pl.ds(j * SUB, SUB), :],
                    sem,
                )
            for j in range(nsub):
                pltpu.make_async_copy(
                    table_hbm.at[idx_v.at[pl.ds(j * SUB, SUB)]],
                    rows_v.at[pl.ds(j * SUB, SUB), :],
                    sem,
                ).wait()

            def tr_body(bg, carry2):
                row_ids = bg * LANES + lax.iota(jnp.int32, LANES)
                vals = [
                    plsc.load_gather(
                        rows_v, [row_ids, jnp.full((LANES,), e, jnp.int32)]
                    )
                    for e in range(EMBED)
                ]
                for e in range(EMBED):
                    slab_v[e, pl.ds(bg * LANES, LANES)] = vals[e]
                return carry2

            lax.fori_loop(0, bw // LANES, tr_body, 0)
            pltpu.sync_copy(slab_v, out_hbm.at[h, :, pl.ds(b0, bw)])
            return carry

        lax.fori_loop(0, hist, body, 0)

    return gather_kernel


def kernel(input_ids, weight):
    batch, hist = input_ids.shape
    ids_t = input_ids.T.astype(jnp.int32)
    out_t = _make_gather(batch, hist)(weight, ids_t)
    return jnp.transpose(out_t, (2, 0, 1))


# speedup vs baseline: 1.3974x; 1.0211x over previous
import functools

import jax
import jax.numpy as jnp
from jax import lax
from jax.experimental import pallas as pl
from jax.experimental.pallas import tpu as pltpu
from jax.experimental.pallas import tpu_sc as plsc

EMBED = 32
LANES = 16
SUB = 128


@functools.lru_cache(maxsize=None)
def _make_gather(batch: int, hist: int):
    info = plsc.get_sparse_core_info()
    nc, ns = info.num_cores, info.num_subcores
    nw = nc * ns
    bw = batch // nw
    nsub = bw // SUB
    assert batch % nw == 0 and bw % SUB == 0
    mesh = plsc.VectorSubcoreMesh(core_axis_name="c", subcore_axis_name="s")

    @functools.partial(
        pl.kernel,
        mesh=mesh,
        out_type=jax.ShapeDtypeStruct((hist, EMBED, batch), jnp.float32),
        scratch_types=[
            pltpu.VMEM((bw,), jnp.int32),
            pltpu.VMEM((bw, EMBED), jnp.float32),
            pltpu.VMEM((EMBED, bw), jnp.float32),
            pltpu.SemaphoreType.DMA,
        ],
        compiler_params=pltpu.CompilerParams(
            use_tc_tiling_on_sc=False, needs_layout_passes=False
        ),
    )
    def gather_kernel(table_hbm, idst_hbm, out_hbm, idx_v, rows_v, slab_v,
                      sem):
        wid = lax.axis_index("s") * nc + lax.axis_index("c")
        b0 = wid * bw
        lane_iota = lax.iota(jnp.int32, LANES)
        col_ids_all = [jnp.full((LANES,), e, jnp.int32) for e in range(EMBED)]

        def body(h, carry):
            pltpu.sync_copy(idst_hbm.at[h, pl.ds(b0, bw)], idx_v)
            for j in range(nsub):
                pltpu.async_copy(
                    table_hbm.at[idx_v.at[pl.ds(j * SUB, SUB)]],
                    rows_v.at[pl.ds(j * SUB, SUB), :],
                    sem,
                )
            for j in range(nsub):
                pltpu.make_async_copy(
                    table_hbm.at[idx_v.at[pl.ds(j * SUB, SUB)]],
                    rows_v.at[pl.ds(j * SUB, SUB), :],
                    sem,
                ).wait()

            @plsc.parallel_loop(0, bw // LANES, 1, unroll=2)
            def tr_body(bg):
                row_ids = bg * LANES + lane_iota
                vals = [
                    plsc.load_gather(rows_v, [row_ids, col_ids_all[e]])
                    for e in range(EMBED)
                ]
                for e in range(EMBED):
                    slab_v[e, pl.ds(bg * LANES, LANES)] = vals[e]
            pltpu.sync_copy(slab_v, out_hbm.at[h, :, pl.ds(b0, bw)])
            return carry

        lax.fori_loop(0, hist, body, 0)

    return gather_kernel


def kernel(input_ids, weight):
    batch, hist = input_ids.shape
    ids_t = input_ids.T.astype(jnp.int32)
    out_t = _make_gather(batch, hist)(weight, ids_t)
    return jnp.transpose(out_t, (2, 0, 1))


# speedup vs baseline: 1.5892x; 1.1373x over previous
import functools

import jax
import jax.numpy as jnp
from jax import lax
from jax.experimental import pallas as pl
from jax.experimental.pallas import tpu as pltpu
from jax.experimental.pallas import tpu_sc as plsc

EMBED = 32
LANES = 16
SUB = 128


@functools.lru_cache(maxsize=None)
def _make_gather(batch: int, hist: int):
    info = plsc.get_sparse_core_info()
    nc, ns = info.num_cores, info.num_subcores
    nw = nc * ns
    bw = batch // nw
    nsub = bw // SUB
    assert batch % nw == 0 and bw % SUB == 0
    mesh = plsc.VectorSubcoreMesh(core_axis_name="c", subcore_axis_name="s")

    @functools.partial(
        pl.kernel,
        mesh=mesh,
        out_type=jax.ShapeDtypeStruct((hist, batch, EMBED), jnp.float32),
        scratch_types=[
            pltpu.VMEM((bw,), jnp.int32),
            pltpu.VMEM((bw, EMBED), jnp.float32),
            pltpu.SemaphoreType.DMA,
        ],
        compiler_params=pltpu.CompilerParams(
            use_tc_tiling_on_sc=False, needs_layout_passes=False
        ),
    )
    def gather_kernel(table_hbm, idst_hbm, out_hbm, idx_v, rows_v, sem):
        wid = lax.axis_index("s") * nc + lax.axis_index("c")
        b0 = wid * bw

        def body(h, carry):
            pltpu.sync_copy(idst_hbm.at[h, pl.ds(b0, bw)], idx_v)
            for j in range(nsub):
                pltpu.async_copy(
                    table_hbm.at[idx_v.at[pl.ds(j * SUB, SUB)]],
                    rows_v.at[pl.ds(j * SUB, SUB), :],
                    sem,
                )
            for j in range(nsub):
                pltpu.make_async_copy(
                    table_hbm.at[idx_v.at[pl.ds(j * SUB, SUB)]],
                    rows_v.at[pl.ds(j * SUB, SUB), :],
                    sem,
                ).wait()

            pltpu.sync_copy(rows_v, out_hbm.at[h, pl.ds(b0, bw)])
            return carry

        lax.fori_loop(0, hist, body, 0)

    return gather_kernel


def kernel(input_ids, weight):
    batch, hist = input_ids.shape
    ids_t = input_ids.T.astype(jnp.int32)
    out_t = _make_gather(batch, hist)(weight, ids_t)
    return jnp.transpose(out_t, (1, 0, 2))


# speedup vs baseline: 1.6123x; 1.0145x over previous
import functools

import jax
import jax.numpy as jnp
from jax import lax
from jax.experimental import pallas as pl
from jax.experimental.pallas import tpu as pltpu
from jax.experimental.pallas import tpu_sc as plsc

EMBED = 32
LANES = 16
SUB = 128


@functools.lru_cache(maxsize=None)
def _make_gather(batch: int, hist: int):
    info = plsc.get_sparse_core_info()
    nc, ns = info.num_cores, info.num_subcores
    nw = nc * ns
    bw = batch // nw
    nsub = bw // SUB
    assert batch % nw == 0 and bw % SUB == 0
    mesh = plsc.VectorSubcoreMesh(core_axis_name="c", subcore_axis_name="s")

    @functools.partial(
        pl.kernel,
        mesh=mesh,
        out_type=jax.ShapeDtypeStruct((hist, EMBED, batch), jnp.float32),
        scratch_types=[
            pltpu.VMEM((bw,), jnp.int32),
            pltpu.VMEM((bw, EMBED), jnp.float32),
            pltpu.VMEM((EMBED, bw), jnp.float32),
            pltpu.SemaphoreType.DMA,
        ],
        compiler_params=pltpu.CompilerParams(
            use_tc_tiling_on_sc=False, needs_layout_passes=False
        ),
    )
    def gather_kernel(table_hbm, idst_hbm, out_hbm, idx_v, rows_v, slab_v,
                      sem):
        wid = lax.axis_index("s") * nc + lax.axis_index("c")
        b0 = wid * bw
        lane_iota = lax.iota(jnp.int32, LANES)
        diag_ids = [(lane_iota + d) & (EMBED - 1) for d in range(EMBED)]

        def body(h, carry):
            pltpu.sync_copy(idst_hbm.at[h, pl.ds(b0, bw)], idx_v)
            for j in range(nsub):
                pltpu.async_copy(
                    table_hbm.at[idx_v.at[pl.ds(j * SUB, SUB)]],
                    rows_v.at[pl.ds(j * SUB, SUB), :],
                    sem,
                )
            for j in range(nsub):
                pltpu.make_async_copy(
                    table_hbm.at[idx_v.at[pl.ds(j * SUB, SUB)]],
                    rows_v.at[pl.ds(j * SUB, SUB), :],
                    sem,
                ).wait()

            @plsc.parallel_loop(0, bw // LANES, 1, unroll=2)
            def tr_body(bg):
                row_ids = bg * LANES + lane_iota
                vals = [
                    plsc.load_gather(rows_v, [row_ids, diag_ids[d]])
                    for d in range(EMBED)
                ]
                for d in range(EMBED):
                    plsc.store_scatter(slab_v, [diag_ids[d], row_ids], vals[d])
            pltpu.sync_copy(slab_v, out_hbm.at[h, :, pl.ds(b0, bw)])
            return carry

        lax.fori_loop(0, hist, body, 0)

    return gather_kernel


def kernel(input_ids, weight):
    batch, hist = input_ids.shape
    ids_t = input_ids.T.astype(jnp.int32)
    out_t = _make_gather(batch, hist)(weight, ids_t)
    return jnp.transpose(out_t, (2, 0, 1))


# speedup vs baseline: 1.9850x; 1.2312x over previous
import functools

import jax
import jax.numpy as jnp
from jax import lax
from jax.experimental import pallas as pl
from jax.experimental.pallas import tpu as pltpu
from jax.experimental.pallas import tpu_sc as plsc

EMBED = 32
LANES = 16
SUB = 128


@functools.lru_cache(maxsize=None)
def _make_gather(batch: int, hist: int):
    info = plsc.get_sparse_core_info()
    nc, ns = info.num_cores, info.num_subcores
    nw = nc * ns
    bw = batch // nw
    nsub = bw // SUB
    assert batch % nw == 0 and bw % SUB == 0
    mesh = plsc.VectorSubcoreMesh(core_axis_name="c", subcore_axis_name="s")

    @functools.partial(
        pl.kernel,
        mesh=mesh,
        out_type=jax.ShapeDtypeStruct((hist, EMBED, batch), jnp.float32),
        scratch_types=[
            pltpu.VMEM((bw,), jnp.int32),
            pltpu.VMEM((bw, EMBED), jnp.float32),
            pltpu.VMEM((EMBED, bw), jnp.float32),
            pltpu.SemaphoreType.DMA,
        ],
        compiler_params=pltpu.CompilerParams(
            use_tc_tiling_on_sc=False, needs_layout_passes=False
        ),
    )
    def gather_kernel(table_hbm, idst_hbm, out_hbm, idx_v, rows_v, slab_v,
                      sem):
        wid = lax.axis_index("s") * nc + lax.axis_index("c")
        b0 = wid * bw
        lane_iota = lax.iota(jnp.int32, LANES)
        diag_ids = [(lane_iota + d) & (EMBED - 1) for d in range(EMBED)]

        def body(h, carry):
            pltpu.sync_copy(idst_hbm.at[h, pl.ds(b0, bw)], idx_v)
            for j in range(nsub):
                pltpu.async_copy(
                    table_hbm.at[idx_v.at[pl.ds(j * SUB, SUB)]],
                    rows_v.at[pl.ds(j * SUB, SUB), :],
                    sem,
                )
            for j in range(nsub):
                pltpu.make_async_copy(
                    table_hbm.at[idx_v.at[pl.ds(j * SUB, SUB)]],
                    rows_v.at[pl.ds(j * SUB, SUB), :],
                    sem,
                ).wait()

            @plsc.parallel_loop(0, bw // LANES, 1, unroll=2)
            def tr_body(bg):
                row_ids = bg * LANES + lane_iota
                for blk in range(0, EMBED, 8):
                    vals = [
                        plsc.load_gather(rows_v, [row_ids, diag_ids[d]])
                        for d in range(blk, blk + 8)
                    ]
                    for i, d in enumerate(range(blk, blk + 8)):
                        plsc.store_scatter(
                            slab_v, [diag_ids[d], row_ids], vals[i]
                        )
            pltpu.sync_copy(slab_v, out_hbm.at[h, :, pl.ds(b0, bw)])
            return carry

        lax.fori_loop(0, hist, body, 0)

    return gather_kernel


def kernel(input_ids, weight):
    batch, hist = input_ids.shape
    ids_t = input_ids.T.astype(jnp.int32)
    out_t = _make_gather(batch, hist)(weight, ids_t)
    return jnp.transpose(out_t, (2, 0, 1))


# speedup vs baseline: 2.7350x; 1.3779x over previous
import functools

import jax
import jax.numpy as jnp
from jax import lax
from jax.experimental import pallas as pl
from jax.experimental.pallas import tpu as pltpu
from jax.experimental.pallas import tpu_sc as plsc

EMBED = 32
LANES = 16
SUB = 128
TBLK = 8


@functools.lru_cache(maxsize=None)
def _make_gather(batch: int, hist: int):
    info = plsc.get_sparse_core_info()
    nc, ns = info.num_cores, info.num_subcores
    nw = nc * ns
    bw = batch // nw
    nsub = bw // SUB
    assert batch % nw == 0 and bw % SUB == 0 and hist % 2 == 0
    mesh = plsc.VectorSubcoreMesh(core_axis_name="c", subcore_axis_name="s")

    @functools.partial(
        pl.kernel,
        mesh=mesh,
        out_type=jax.ShapeDtypeStruct((hist, EMBED, batch), jnp.float32),
        scratch_types=[
            pltpu.VMEM((2, bw), jnp.int32),
            pltpu.VMEM((2, bw, EMBED), jnp.float32),
            pltpu.VMEM((2, EMBED, bw), jnp.float32),
            pltpu.SemaphoreType.DMA,
            pltpu.SemaphoreType.DMA,
            pltpu.SemaphoreType.DMA,
            pltpu.SemaphoreType.DMA,
            pltpu.SemaphoreType.DMA,
            pltpu.SemaphoreType.DMA,
        ],
        compiler_params=pltpu.CompilerParams(
            use_tc_tiling_on_sc=False, needs_layout_passes=False
        ),
    )
    def gather_kernel(table_hbm, idst_hbm, out_hbm, idx_v, rows_v, slab_v,
                      g_a, g_b, i_a, i_b, o_a, o_b):
        wid = lax.axis_index("s") * nc + lax.axis_index("c")
        b0 = wid * bw
        lane_iota = lax.iota(jnp.int32, LANES)
        diag_ids = [(lane_iota + d) & (EMBED - 1) for d in range(EMBED)]

        def fire_gathers(p, sem):
            for j in range(nsub):
                pltpu.async_copy(
                    table_hbm.at[idx_v.at[p, pl.ds(j * SUB, SUB)]],
                    rows_v.at[p, pl.ds(j * SUB, SUB), :],
                    sem,
                )

        def drain_gathers(p, sem):
            for j in range(nsub):
                pltpu.make_async_copy(
                    table_hbm.at[idx_v.at[p, pl.ds(j * SUB, SUB)]],
                    rows_v.at[p, pl.ds(j * SUB, SUB), :],
                    sem,
                ).wait()

        def idx_fetch(h, p, sem):
            pltpu.async_copy(idst_hbm.at[h, pl.ds(b0, bw)], idx_v.at[p], sem)

        def idx_wait(h, p, sem):
            pltpu.make_async_copy(
                idst_hbm.at[h, pl.ds(b0, bw)], idx_v.at[p], sem
            ).wait()

        def transpose(p):
            @plsc.parallel_loop(0, bw // LANES, 1, unroll=2)
            def tr_body(bg):
                row_ids = bg * LANES + lane_iota
                for blk in range(0, EMBED, TBLK):
                    vals = [
                        plsc.load_gather(
                            rows_v.at[p], [row_ids, diag_ids[d]]
                        )
                        for d in range(blk, blk + TBLK)
                    ]
                    for i, d in enumerate(range(blk, blk + TBLK)):
                        plsc.store_scatter(
                            slab_v.at[p], [diag_ids[d], row_ids], vals[i]
                        )

        def out_write(h, p, sem):
            pltpu.async_copy(
                slab_v.at[p], out_hbm.at[h, :, pl.ds(b0, bw)], sem
            )

        def out_wait(h, p, sem):
            pltpu.make_async_copy(
                slab_v.at[p], out_hbm.at[h, :, pl.ds(b0, bw)], sem
            ).wait()

        pltpu.sync_copy(idst_hbm.at[0, pl.ds(b0, bw)], idx_v.at[0])
        fire_gathers(0, g_a)
        idx_fetch(1, 1, i_b)

        def body(hh, carry):
            h0 = 2 * hh
            h1 = h0 + 1
            drain_gathers(0, g_a)
            idx_wait(h1, 1, i_b)
            fire_gathers(1, g_b)

            @pl.when(hh < hist // 2 - 1)
            def _():
                idx_fetch(h0 + 2, 0, i_a)

            @pl.when(hh > 0)
            def _():
                out_wait(h0, 0, o_a)

            transpose(0)
            out_write(h0, 0, o_a)

            drain_gathers(1, g_b)

            @pl.when(hh < hist // 2 - 1)
            def _():
                idx_wait(h0 + 2, 0, i_a)
                fire_gathers(0, g_a)
                idx_fetch(h0 + 3, 1, i_b)

            @pl.when(hh > 0)
            def _():
                out_wait(h1, 1, o_b)

            transpose(1)
            out_write(h1, 1, o_b)
            return carry

        lax.fori_loop(0, hist // 2, body, 0)
        out_wait(hist - 2, 0, o_a)
        out_wait(hist - 1, 1, o_b)

    return gather_kernel


def kernel(input_ids, weight):
    batch, hist = input_ids.shape
    ids_t = input_ids.T.astype(jnp.int32)
    out_t = _make_gather(batch, hist)(weight, ids_t)
    return jnp.transpose(out_t, (2, 0, 1))


# speedup vs baseline: 4.1957x; 1.5341x over previous
import functools

import jax
import jax.numpy as jnp
from jax import lax
from jax.experimental import pallas as pl
from jax.experimental.pallas import tpu as pltpu
from jax.experimental.pallas import tpu_sc as plsc

EMBED = 32
LANES = 16
SUB = 128
TBLK = 8


@functools.lru_cache(maxsize=None)
def _make_gather(batch: int, hist: int):
    info = plsc.get_sparse_core_info()
    nc, ns = info.num_cores, info.num_subcores
    nw = nc * ns
    bw = batch // nw
    nsub = bw // SUB
    assert batch % nw == 0 and bw % SUB == 0 and hist % 2 == 0
    mesh = plsc.VectorSubcoreMesh(core_axis_name="c", subcore_axis_name="s")

    @functools.partial(
        pl.kernel,
        mesh=mesh,
        out_type=jax.ShapeDtypeStruct(
            (hist, EMBED // 8, batch // 128, 8, 128), jnp.float32
        ),
        scratch_types=[
            pltpu.VMEM((2, bw), jnp.int32),
            pltpu.VMEM((2, bw, EMBED), jnp.float32),
            pltpu.VMEM((2, EMBED // 8, bw // 128, 8, 128), jnp.float32),
            pltpu.SemaphoreType.DMA,
            pltpu.SemaphoreType.DMA,
            pltpu.SemaphoreType.DMA,
            pltpu.SemaphoreType.DMA,
            pltpu.SemaphoreType.DMA,
            pltpu.SemaphoreType.DMA,
        ],
        compiler_params=pltpu.CompilerParams(
            use_tc_tiling_on_sc=False, needs_layout_passes=False
        ),
    )
    def gather_kernel(table_hbm, idst_hbm, out_hbm, idx_v, rows_v, slab_v,
                      g_a, g_b, i_a, i_b, o_a, o_b):
        wid = lax.axis_index("s") * nc + lax.axis_index("c")
        b0 = wid * bw
        lane_iota = lax.iota(jnp.int32, LANES)
        diag_ids = [(lane_iota + d) & (EMBED - 1) for d in range(EMBED)]

        def fire_gathers(p, sem):
            for j in range(nsub):
                pltpu.async_copy(
                    table_hbm.at[idx_v.at[p, pl.ds(j * SUB, SUB)]],
                    rows_v.at[p, pl.ds(j * SUB, SUB), :],
                    sem,
                )

        def drain_gathers(p, sem):
            for j in range(nsub):
                pltpu.make_async_copy(
                    table_hbm.at[idx_v.at[p, pl.ds(j * SUB, SUB)]],
                    rows_v.at[p, pl.ds(j * SUB, SUB), :],
                    sem,
                ).wait()

        def idx_fetch(h, p, sem):
            pltpu.async_copy(idst_hbm.at[h, pl.ds(b0, bw)], idx_v.at[p], sem)

        def idx_wait(h, p, sem):
            pltpu.make_async_copy(
                idst_hbm.at[h, pl.ds(b0, bw)], idx_v.at[p], sem
            ).wait()

        def transpose(p):
            @plsc.parallel_loop(0, bw // LANES, 1, unroll=2)
            def tr_body(bg):
                row_ids = bg * LANES + lane_iota
                btl = lax.shift_right_logical(row_ids, 7)
                bi = lax.bitwise_and(row_ids, 127)
                for blk in range(0, EMBED, TBLK):
                    vals = [
                        plsc.load_gather(
                            rows_v.at[p], [row_ids, diag_ids[d]]
                        )
                        for d in range(blk, blk + TBLK)
                    ]
                    for i, d in enumerate(range(blk, blk + TBLK)):
                        et = lax.shift_right_logical(diag_ids[d], 3)
                        ei = lax.bitwise_and(diag_ids[d], 7)
                        plsc.store_scatter(
                            slab_v.at[p], [et, btl, ei, bi], vals[i]
                        )

        def out_write(h, p, sem):
            pltpu.async_copy(
                slab_v.at[p],
                out_hbm.at[h, :, pl.ds(wid * (bw // 128), bw // 128)],
                sem,
            )

        def out_wait(h, p, sem):
            pltpu.make_async_copy(
                slab_v.at[p],
                out_hbm.at[h, :, pl.ds(wid * (bw // 128), bw // 128)],
                sem,
            ).wait()

        pltpu.sync_copy(idst_hbm.at[0, pl.ds(b0, bw)], idx_v.at[0])
        fire_gathers(0, g_a)
        idx_fetch(1, 1, i_b)

        def body(hh, carry):
            h0 = 2 * hh
            h1 = h0 + 1
            drain_gathers(0, g_a)
            idx_wait(h1, 1, i_b)
            fire_gathers(1, g_b)

            @pl.when(hh < hist // 2 - 1)
            def _():
                idx_fetch(h0 + 2, 0, i_a)

            @pl.when(hh > 0)
            def _():
                out_wait(h0, 0, o_a)

            transpose(0)
            out_write(h0, 0, o_a)

            drain_gathers(1, g_b)

            @pl.when(hh < hist // 2 - 1)
            def _():
                idx_wait(h0 + 2, 0, i_a)
                fire_gathers(0, g_a)
                idx_fetch(h0 + 3, 1, i_b)

            @pl.when(hh > 0)
            def _():
                out_wait(h1, 1, o_b)

            transpose(1)
            out_write(h1, 1, o_b)
            return carry

        lax.fori_loop(0, hist // 2, body, 0)
        out_wait(hist - 2, 0, o_a)
        out_wait(hist - 1, 1, o_b)

    return gather_kernel


def kernel(input_ids, weight):
    batch, hist = input_ids.shape
    ids_t = input_ids.T.astype(jnp.int32)
    out5 = _make_gather(batch, hist)(weight, ids_t)
    out = jnp.transpose(out5, (2, 4, 0, 1, 3))
    return out.reshape(batch, hist, 32)


# speedup vs baseline: 4.6190x; 1.1009x over previous
import functools

import jax
import jax.numpy as jnp
from jax import lax
from jax.experimental import pallas as pl
from jax.experimental.pallas import tpu as pltpu
from jax.experimental.pallas import tpu_sc as plsc

EMBED = 32
LANES = 16
SUB = 128
TBLK = 16


@functools.lru_cache(maxsize=None)
def _make_gather(batch: int, hist: int):
    info = plsc.get_sparse_core_info()
    nc, ns = info.num_cores, info.num_subcores
    nw = nc * ns
    bw = batch // nw
    nsub = bw // SUB
    assert batch % nw == 0 and bw % SUB == 0 and hist % 2 == 0
    mesh = plsc.VectorSubcoreMesh(core_axis_name="c", subcore_axis_name="s")

    @functools.partial(
        pl.kernel,
        mesh=mesh,
        out_type=jax.ShapeDtypeStruct(
            (hist, EMBED // 8, batch // 128, 8, 128), jnp.float32
        ),
        scratch_types=[
            pltpu.VMEM((2, bw), jnp.int32),
            pltpu.VMEM((2, bw, EMBED), jnp.float32),
            pltpu.VMEM((2, EMBED // 8, bw // 128, 8, 128), jnp.float32),
            pltpu.SemaphoreType.DMA,
            pltpu.SemaphoreType.DMA,
            pltpu.SemaphoreType.DMA,
            pltpu.SemaphoreType.DMA,
            pltpu.SemaphoreType.DMA,
            pltpu.SemaphoreType.DMA,
        ],
        compiler_params=pltpu.CompilerParams(
            use_tc_tiling_on_sc=False, needs_layout_passes=False
        ),
    )
    def gather_kernel(table_hbm, idst_hbm, out_hbm, idx_v, rows_v, slab_v,
                      g_a, g_b, i_a, i_b, o_a, o_b):
        wid = lax.axis_index("s") * nc + lax.axis_index("c")
        b0 = wid * bw
        lane_iota = lax.iota(jnp.int32, LANES)
        diag_ids = [(lane_iota + d) & (EMBED - 1) for d in range(EMBED)]

        def fire_gathers(p, sem):
            for j in range(nsub):
                pltpu.async_copy(
                    table_hbm.at[idx_v.at[p, pl.ds(j * SUB, SUB)]],
                    rows_v.at[p, pl.ds(j * SUB, SUB), :],
                    sem,
                )

        def drain_gathers(p, sem):
            for j in range(nsub):
                pltpu.make_async_copy(
                    table_hbm.at[idx_v.at[p, pl.ds(j * SUB, SUB)]],
                    rows_v.at[p, pl.ds(j * SUB, SUB), :],
                    sem,
                ).wait()

        def idx_fetch(h, p, sem):
            pltpu.async_copy(idst_hbm.at[h, pl.ds(b0, bw)], idx_v.at[p], sem)

        def idx_wait(h, p, sem):
            pltpu.make_async_copy(
                idst_hbm.at[h, pl.ds(b0, bw)], idx_v.at[p], sem
            ).wait()

        def transpose(p):
            @plsc.parallel_loop(0, bw // LANES, 1, unroll=2)
            def tr_body(bg):
                row_ids = bg * LANES + lane_iota
                btl = lax.shift_right_logical(row_ids, 7)
                bi = lax.bitwise_and(row_ids, 127)
                for blk in range(0, EMBED, TBLK):
                    vals = [
                        plsc.load_gather(
                            rows_v.at[p], [row_ids, diag_ids[d]]
                        )
                        for d in range(blk, blk + TBLK)
                    ]
                    for i, d in enumerate(range(blk, blk + TBLK)):
                        et = lax.shift_right_logical(diag_ids[d], 3)
                        ei = lax.bitwise_and(diag_ids[d], 7)
                        plsc.store_scatter(
                            slab_v.at[p], [et, btl, ei, bi], vals[i]
                        )

        def out_write(h, p, sem):
            pltpu.async_copy(
                slab_v.at[p],
                out_hbm.at[h, :, pl.ds(wid * (bw // 128), bw // 128)],
                sem,
            )

        def out_wait(h, p, sem):
            pltpu.make_async_copy(
                slab_v.at[p],
                out_hbm.at[h, :, pl.ds(wid * (bw // 128), bw // 128)],
                sem,
            ).wait()

        pltpu.sync_copy(idst_hbm.at[0, pl.ds(b0, bw)], idx_v.at[0])
        fire_gathers(0, g_a)
        idx_fetch(1, 1, i_b)

        def body(hh, carry):
            h0 = 2 * hh
            h1 = h0 + 1
            drain_gathers(0, g_a)
            idx_wait(h1, 1, i_b)
            fire_gathers(1, g_b)

            @pl.when(hh < hist // 2 - 1)
            def _():
                idx_fetch(h0 + 2, 0, i_a)

            @pl.when(hh > 0)
            def _():
                out_wait(h0, 0, o_a)

            transpose(0)
            out_write(h0, 0, o_a)

            drain_gathers(1, g_b)

            @pl.when(hh < hist // 2 - 1)
            def _():
                idx_wait(h0 + 2, 0, i_a)
                fire_gathers(0, g_a)
                idx_fetch(h0 + 3, 1, i_b)

            @pl.when(hh > 0)
            def _():
                out_wait(h1, 1, o_b)

            transpose(1)
            out_write(h1, 1, o_b)
            return carry

        lax.fori_loop(0, hist // 2, body, 0)
        out_wait(hist - 2, 0, o_a)
        out_wait(hist - 1, 1, o_b)

    return gather_kernel


def kernel(input_ids, weight):
    batch, hist = input_ids.shape
    ids_t = input_ids.T.astype(jnp.int32)
    out5 = _make_gather(batch, hist)(weight, ids_t)
    out = jnp.transpose(out5, (2, 4, 0, 1, 3))
    return out.reshape(batch, hist, 32)
